# baseline (speedup 1.0000x reference)
import functools
import jax
import jax.numpy as jnp
from jax.experimental import pallas as pl
from jax.experimental.pallas import tpu as pltpu

MAX_OUT = 300
IOU_T = 0.7
SCORE_T = 0.0
PROP_T = 0.5
LANES = 128
SPEC = 8


def _nms_body(score_ref, delta_ref, anch_ref,
              kx0_ref, kx1_ref, ky0_ref, ky1_ref,
              sc0_ref, bx00, bx10, by00, by10,
              sc1_ref, bx01, bx11, by01, by11):
    B = 2
    scs = (sc0_ref, sc1_ref)
    bxs = ((bx00, bx10, by00, by10), (bx01, bx11, by01, by11))
    R = sc0_ref.shape[0]

    rowio = jax.lax.broadcasted_iota(jnp.int32, (R, LANES), 0)
    laneio = jax.lax.broadcasted_iota(jnp.int32, (1, LANES), 1)
    flat8 = (jax.lax.broadcasted_iota(jnp.int32, (8, LANES), 0) * LANES
             + jax.lax.broadcasted_iota(jnp.int32, (8, LANES), 1))
    u2d = jax.lax.broadcasted_iota(jnp.int32, (LANES, LANES), 0)
    v2d = jax.lax.broadcasted_iota(jnp.int32, (LANES, LANES), 1)

    for i in range(B):
        a0 = anch_ref[0]
        a1 = anch_ref[1]
        a2 = anch_ref[2]
        a3 = anch_ref[3]
        xa = (a0 + a1) * 0.5
        ya = (a2 + a3) * 0.5
        wa = a1 - a0
        ha = a3 - a2
        tx = delta_ref[i, 0]
        ty = delta_ref[i, 1]
        tw = delta_ref[i, 2]
        th = delta_ref[i, 3]
        x = tx * wa + xa
        y = ty * ha + ya
        w = jnp.exp(tw) * wa
        h = jnp.exp(th) * ha
        bx0_r, bx1_r, by0_r, by1_r = bxs[i]
        bx0_r[...] = jnp.clip(x - w * 0.5, 0.0, 1.0)
        bx1_r[...] = jnp.clip(x + w * 0.5, 0.0, 1.0)
        by0_r[...] = jnp.clip(y - h * 0.5, 0.0, 1.0)
        by1_r[...] = jnp.clip(y + h * 0.5, 0.0, 1.0)
        s = score_ref[i]
        scs[i][...] = jnp.where(s > PROP_T, s, -1.0)

    kx0_ref[...] = jnp.zeros_like(kx0_ref)
    kx1_ref[...] = jnp.zeros_like(kx1_ref)
    ky0_ref[...] = jnp.zeros_like(ky0_ref)
    ky1_ref[...] = jnp.zeros_like(ky1_ref)

    def precedes(a, b):
        sa, ra, la = a
        sb, rb, lb = b
        return jnp.logical_or(
            sa > sb,
            jnp.logical_and(sa == sb, jnp.logical_or(
                ra < rb, jnp.logical_and(ra == rb, la < lb))))

    def iou_pair(a, b):
        ax0, ax1, ay0, ay1 = a
        bx0, bx1, by0, by1 = b
        iw = jnp.maximum(jnp.minimum(ax1, bx1) - jnp.maximum(ax0, bx0), 0.0)
        ih = jnp.maximum(jnp.minimum(ay1, by1) - jnp.maximum(ay0, by0), 0.0)
        inter = iw * ih
        area_a = (ax1 - ax0) * (ay1 - ay0)
        area_b = (bx1 - bx0) * (by1 - by0)
        return inter / (area_a + area_b - inter + 1e-9)

    def pack(cm, cr):
        bits = jax.lax.bitcast_convert_type(cm, jnp.int32)
        key = ((bits & 0x7FFFFF) << 8) | (R - 1 - cr)
        return jnp.where(cm > PROP_T, key, 0)

    def phase(i, k):
        sc_ref = scs[i]
        bx0_r, bx1_r, by0_r, by1_r = bxs[i]
        sc = sc_ref[...]

        colmax = jnp.max(sc, axis=0, keepdims=True)
        colrow = jnp.min(jnp.where(sc == colmax, rowio, R), axis=0,
                         keepdims=True)
        m_all = jnp.max(colmax, axis=1, keepdims=True)

        champ = rowio == colrow
        sc2 = jnp.where(champ, -1.0, sc)
        colmax2 = jnp.max(sc2, axis=0, keepdims=True)
        colrow2 = jnp.min(jnp.where(sc2 == colmax2, rowio, R), axis=0,
                          keepdims=True)

        key1 = pack(colmax, colrow)
        keyT = jnp.swapaxes(key1, 0, 1)
        pre = jnp.logical_or(keyT > key1,
                             jnp.logical_and(keyT == key1, u2d < v2d))
        rank = jnp.sum(pre.astype(jnp.int32), axis=0, keepdims=True)

        kx0 = kx0_ref[i]
        kx1 = kx1_ref[i]
        ky0 = ky0_ref[i]
        ky1 = ky1_ref[i]

        one = jnp.ones((1, 1), jnp.bool_)
        prefix = one
        thr = (jnp.full((1, 1), -2.0), jnp.full((1, 1), R, jnp.int32),
               jnp.full((1, 1), LANES, jnp.int32))
        mates = []
        for t in range(SPEC):
            lmask = rank == t
            s_t = jnp.max(jnp.where(lmask, colmax, -2.0), axis=1,
                          keepdims=True)
            r_t = jnp.max(jnp.where(lmask, colrow, -1), axis=1,
                          keepdims=True)
            l_t = jnp.max(jnp.where(lmask, laneio, -1), axis=1,
                          keepdims=True)
            s2_t = jnp.max(jnp.where(lmask, colmax2, -2.0), axis=1,
                           keepdims=True)
            r2_t = jnp.max(jnp.where(lmask, colrow2, -1), axis=1,
                           keepdims=True)

            prefix = jnp.logical_and(
                prefix, jnp.logical_not(precedes(thr, (s_t, r_t, l_t))))

            row_sc = jnp.max(r_t)
            srow = sc_ref[pl.ds(row_sc, 1), :]
            sc_ref[pl.ds(row_sc, 1), :] = jnp.where(
                jnp.logical_and(lmask, prefix), -1.0, srow)

            rows4 = jnp.concatenate(
                [bx0_r[pl.ds(row_sc, 1), :], bx1_r[pl.ds(row_sc, 1), :],
                 by0_r[pl.ds(row_sc, 1), :], by1_r[pl.ds(row_sc, 1), :]],
                axis=0)
            coords = jnp.max(jnp.where(lmask, rows4, -1.0), axis=1,
                             keepdims=True)
            box = (coords[0:1], coords[1:2], coords[2:3], coords[3:4])
            x0, x1, y0, y1 = box

            iw = jnp.maximum(jnp.minimum(x1, kx1) - jnp.maximum(x0, kx0), 0.0)
            ih = jnp.maximum(jnp.minimum(y1, ky1) - jnp.maximum(y0, ky0), 0.0)
            inter = iw * ih
            area = (x1 - x0) * (y1 - y0)
            areas = (kx1 - kx0) * (ky1 - ky0)
            iou = inter / (area + areas - inter + 1e-9)
            ovf = jnp.max(jnp.where(iou > IOU_T, 1.0, 0.0), axis=0,
                          keepdims=True)
            ovk = jnp.max(ovf, axis=1, keepdims=True) > 0.5

            ovb = jnp.zeros((1, 1), jnp.bool_)
            for mbox, mkeep in mates:
                hit = jnp.logical_and(mkeep, iou_pair(mbox, box) > IOU_T)
                ovb = jnp.logical_or(ovb, hit)

            active = jnp.logical_and(
                jnp.logical_and(k < MAX_OUT, s_t > SCORE_T), prefix)
            keep = jnp.logical_and(
                active, jnp.logical_not(jnp.logical_or(ovk, ovb)))

            sel = jnp.logical_and(flat8 == k, keep)
            kx0_ref[i] = jnp.where(sel, x0, kx0_ref[i])
            kx1_ref[i] = jnp.where(sel, x1, kx1_ref[i])
            ky0_ref[i] = jnp.where(sel, y0, ky0_ref[i])
            ky1_ref[i] = jnp.where(sel, y1, ky1_ref[i])
            k = k + keep.astype(jnp.int32)
            mates.append((box, keep))

            t2 = (s2_t, r2_t, l_t)
            upd = jnp.logical_and(prefix, precedes(t2, thr))
            thr = tuple(jnp.where(upd, n, o) for n, o in zip(t2, thr))

        return k, m_all

    def cond(carry):
        k0, m0, k1, m1 = carry
        a0 = jnp.logical_and(k0 < MAX_OUT, m0 > SCORE_T)
        a1 = jnp.logical_and(k1 < MAX_OUT, m1 > SCORE_T)
        return jnp.any(jnp.logical_or(a0, a1))

    def body(carry):
        k0, m0, k1, m1 = carry
        k0, m0 = phase(0, k0)
        k1, m1 = phase(1, k1)
        return k0, m0, k1, m1

    zk = jnp.zeros((1, 1), jnp.int32)
    zm = jnp.ones((1, 1), jnp.float32)
    jax.lax.while_loop(cond, body, (zk, zm, zk, zm))


@functools.partial(jax.jit, static_argnames=("interpret",))
def kernel(score_map, delta_map, anchors, interpret=False):
    B, H, W, A = score_map.shape
    N = H * W * A
    R = N // LANES
    assert N % LANES == 0 and R <= 256 and B == 2

    scores = score_map.reshape(B, R, LANES)
    deltas = delta_map.reshape(B, N, 4).transpose(0, 2, 1).reshape(B, 4, R, LANES)
    anch = anchors.reshape(N, 4).T.reshape(4, R, LANES)

    shp = jax.ShapeDtypeStruct((B, 8, LANES), jnp.float32)
    per_batch = [pltpu.VMEM((R, LANES), jnp.float32)] * 5
    kx0, kx1, ky0, ky1 = pl.pallas_call(
        _nms_body,
        out_shape=(shp, shp, shp, shp),
        scratch_shapes=per_batch + per_batch,
        interpret=interpret,
    )(scores, deltas, anch)
    out = jnp.stack([c.reshape(B, 8 * LANES)[:, :MAX_OUT]
                     for c in (kx0, kx1, ky0, ky1)], axis=-1)
    return out


# speedup vs baseline: 1.6415x; 1.6415x over previous
import functools
import jax
import jax.numpy as jnp
from jax import lax
from jax.experimental import pallas as pl
from jax.experimental.pallas import tpu as pltpu
from jax.experimental.pallas import tpu_sc as plsc

MAX_OUT = 300
IOU_T = 0.7
SCORE_T = 0.0
PROP_T = 0.5
LANES = 128
L = 16
KPAD = 304
UNROLL = 8


def _decode_body(delta_ref, anch_ref, out_ref):
    B = delta_ref.shape[0]
    for i in range(B):
        a0 = anch_ref[0]
        a1 = anch_ref[1]
        a2 = anch_ref[2]
        a3 = anch_ref[3]
        xa = (a0 + a1) * 0.5
        ya = (a2 + a3) * 0.5
        wa = a1 - a0
        ha = a3 - a2
        tx = delta_ref[i, 0]
        ty = delta_ref[i, 1]
        tw = delta_ref[i, 2]
        th = delta_ref[i, 3]
        x = tx * wa + xa
        y = ty * ha + ya
        w = jnp.exp(tw) * wa
        h = jnp.exp(th) * ha
        out_ref[i, 0] = jnp.clip(x - w * 0.5, 0.0, 1.0)
        out_ref[i, 1] = jnp.clip(x + w * 0.5, 0.0, 1.0)
        out_ref[i, 2] = jnp.clip(y - h * 0.5, 0.0, 1.0)
        out_ref[i, 3] = jnp.clip(y + h * 0.5, 0.0, 1.0)


def _make_sc_nms(N):
    G1 = N // L
    NV1 = G1 // L
    NV2 = (NV1 + L - 1) // L

    mesh = plsc.VectorSubcoreMesh(core_axis_name="c", subcore_axis_name="s")

    @functools.partial(
        pl.kernel, mesh=mesh,
        compiler_params=pltpu.CompilerParams(needs_layout_passes=False),
        out_type=jax.ShapeDtypeStruct((2 * 4 * KPAD,), jnp.float32),
        scratch_types=[
            pltpu.VMEM((N,), jnp.float32),
            pltpu.VMEM((N,), jnp.float32),
            pltpu.VMEM((N,), jnp.float32),
            pltpu.VMEM((N,), jnp.float32),
            pltpu.VMEM((N,), jnp.float32),
            pltpu.VMEM((G1,), jnp.float32),
            pltpu.VMEM((LANES,), jnp.float32),
            pltpu.VMEM((KPAD,), jnp.float32),
            pltpu.VMEM((KPAD,), jnp.float32),
            pltpu.VMEM((KPAD,), jnp.float32),
            pltpu.VMEM((KPAD,), jnp.float32),
        ],
    )
    def sc_nms(scores_hbm, boxes_hbm, out_hbm,
               sc_v, bx0_v, bx1_v, by0_v, by1_v, l1_v, l2_v,
               k0_v, k1_v, k2_v, k3_v):
        cid = lax.axis_index("c")
        sid = lax.axis_index("s")

        lanes = jnp.arange(L, dtype=jnp.int32)
        m0 = lanes == 0
        zf = jnp.zeros((L,), jnp.float32)
        zi = jnp.zeros((L,), jnp.int32)
        BIG = jnp.int32(1 << 24)

        def rot(x, sh):
            return x.at[(lanes + sh) % L].get(mode="promise_in_bounds")

        def allmax(x):
            for sh in (1, 2, 4, 8):
                x = jnp.maximum(x, rot(x, sh))
            return x

        def allmin(x):
            for sh in (1, 2, 4, 8):
                x = jnp.minimum(x, rot(x, sh))
            return x

        @pl.when(sid == 0)
        def _():
            i = cid
            pltpu.sync_copy(scores_hbm.at[pl.ds(i * N, N)], sc_v)
            pltpu.sync_copy(boxes_hbm.at[pl.ds((i * 4 + 0) * N, N)], bx0_v)
            pltpu.sync_copy(boxes_hbm.at[pl.ds((i * 4 + 1) * N, N)], bx1_v)
            pltpu.sync_copy(boxes_hbm.at[pl.ds((i * 4 + 2) * N, N)], by0_v)
            pltpu.sync_copy(boxes_hbm.at[pl.ds((i * 4 + 3) * N, N)], by1_v)

            def l1_body(cc, _):
                base = cc * L
                acc = zf - 1.0
                for r in range(L):
                    off = r * G1 + base
                    v = sc_v[pl.ds(off, L)]
                    v = jnp.where(v > PROP_T, v, -1.0)
                    sc_v[pl.ds(off, L)] = v
                    acc = jnp.maximum(acc, v)
                l1_v[pl.ds(base, L)] = acc
                return 0
            lax.fori_loop(0, NV1, l1_body, 0)

            for vi in range(LANES // L):
                l2_v[pl.ds(vi * L, L)] = zf - 1e9

            def l2_body(h, _):
                msp = allmax(l1_v[pl.ds(h * L, L)])
                plsc.store_scatter(l2_v, [zi + h], msp, mask=m0)
                return 0
            lax.fori_loop(0, NV1, l2_body, 0)

            for t in range(KPAD // L):
                k0_v[pl.ds(t * L, L)] = zf
                k1_v[pl.ds(t * L, L)] = zf
                k2_v[pl.ds(t * L, L)] = zf
                k3_v[pl.ds(t * L, L)] = zf

            def topmax():
                acc = l2_v[pl.ds(0, L)]
                for vi in range(1, NV2):
                    acc = jnp.maximum(acc, l2_v[pl.ds(vi * L, L)])
                return allmax(acc)

            def step(k_sp):
                M = topmax()
                alive = jnp.logical_and(k_sp < MAX_OUT, M > SCORE_T)

                hc = zi + BIG
                for vi in range(NV2):
                    mvi = l2_v[pl.ds(vi * L, L)] == M
                    hc = jnp.minimum(hc, jnp.where(mvi, vi * L + lanes, BIG))
                h = allmin(hc)
                g1 = plsc.load_gather(l1_v, [h * L + lanes])
                c = allmin(jnp.where(g1 == M, h * L + lanes, BIG))
                col = plsc.load_gather(sc_v, [lanes * G1 + c])
                r = allmin(jnp.where(col == M, lanes, BIG))
                j = c * L + r

                wmask = jnp.logical_and(m0, alive)
                plsc.store_scatter(sc_v, [r * G1 + c], zf - 1.0, mask=wmask)
                newc = allmax(plsc.load_gather(sc_v, [lanes * G1 + c]))
                plsc.store_scatter(l1_v, [c], newc, mask=wmask)
                newh = allmax(plsc.load_gather(l1_v, [h * L + lanes]))
                plsc.store_scatter(l2_v, [h], newh, mask=wmask)

                x0 = plsc.load_gather(bx0_v, [j])
                x1 = plsc.load_gather(bx1_v, [j])
                y0 = plsc.load_gather(by0_v, [j])
                y1 = plsc.load_gather(by1_v, [j])

                acc = lanes < 0
                for t in range(KPAD // L):
                    b = t * L
                    kx0 = k0_v[pl.ds(b, L)]
                    kx1 = k1_v[pl.ds(b, L)]
                    ky0 = k2_v[pl.ds(b, L)]
                    ky1 = k3_v[pl.ds(b, L)]
                    iw = jnp.maximum(
                        jnp.minimum(x1, kx1) - jnp.maximum(x0, kx0), 0.0)
                    ih = jnp.maximum(
                        jnp.minimum(y1, ky1) - jnp.maximum(y0, ky0), 0.0)
                    inter = iw * ih
                    area = (x1 - x0) * (y1 - y0)
                    areas = (kx1 - kx0) * (ky1 - ky0)
                    iou = inter / (area + areas - inter + 1e-9)
                    acc = jnp.logical_or(acc, iou > IOU_T)
                ov = allmax(jnp.where(acc, 1, 0))
                keep = jnp.logical_and(alive, ov == 0)

                kmask = jnp.logical_and(m0, keep)
                plsc.store_scatter(k0_v, [k_sp], x0, mask=kmask)
                plsc.store_scatter(k1_v, [k_sp], x1, mask=kmask)
                plsc.store_scatter(k2_v, [k_sp], y0, mask=kmask)
                plsc.store_scatter(k3_v, [k_sp], y1, mask=kmask)

                k_sp = k_sp + jnp.where(keep, 1, 0)
                return k_sp, jnp.logical_and(alive, k_sp < MAX_OUT)

            def cond(carry):
                _, flag = carry
                return flag > 0

            def body(carry):
                k_sp, _ = carry
                alive = m0
                for _ in range(UNROLL):
                    k_sp, alive = step(k_sp)
                return k_sp, jnp.where(alive, 1, 0)[0]

            lax.while_loop(cond, body, (zi, jnp.int32(1)))

            pltpu.sync_copy(k0_v, out_hbm.at[pl.ds((i * 4 + 0) * KPAD, KPAD)])
            pltpu.sync_copy(k1_v, out_hbm.at[pl.ds((i * 4 + 1) * KPAD, KPAD)])
            pltpu.sync_copy(k2_v, out_hbm.at[pl.ds((i * 4 + 2) * KPAD, KPAD)])
            pltpu.sync_copy(k3_v, out_hbm.at[pl.ds((i * 4 + 3) * KPAD, KPAD)])

    return sc_nms


@jax.jit
def kernel(score_map, delta_map, anchors):
    B, H, W, A = score_map.shape
    N = H * W * A
    R = N // LANES
    assert N % LANES == 0 and B == 2

    deltas = delta_map.reshape(B, N, 4).transpose(0, 2, 1).reshape(B, 4, R, LANES)
    anch = anchors.reshape(N, 4).T.reshape(4, R, LANES)

    boxes = pl.pallas_call(
        _decode_body,
        out_shape=jax.ShapeDtypeStruct((B, 4, R, LANES), jnp.float32),
    )(deltas, anch)

    scores_t = (score_map.reshape(B, N // L, L)
                .transpose(0, 2, 1).reshape(B * N))
    boxes_flat = boxes.reshape(B * 4 * N)

    out = _make_sc_nms(N)(scores_t, boxes_flat)
    kept = out.reshape(B, 4, KPAD)[:, :, :MAX_OUT]
    return kept.transpose(0, 2, 1)


# speedup vs baseline: 1.7336x; 1.0562x over previous
import functools
import jax
import jax.numpy as jnp
from jax import lax
from jax.experimental import pallas as pl
from jax.experimental.pallas import tpu as pltpu
from jax.experimental.pallas import tpu_sc as plsc

MAX_OUT = 300
IOU_T = 0.7
SCORE_T = 0.0
PROP_T = 0.5
LANES = 128
L = 16
KPAD = 304
UNROLL = 8


def _decode_body(delta_ref, anch_ref, out_ref):
    B = delta_ref.shape[0]
    for i in range(B):
        a0 = anch_ref[0]
        a1 = anch_ref[1]
        a2 = anch_ref[2]
        a3 = anch_ref[3]
        xa = (a0 + a1) * 0.5
        ya = (a2 + a3) * 0.5
        wa = a1 - a0
        ha = a3 - a2
        tx = delta_ref[i, 0]
        ty = delta_ref[i, 1]
        tw = delta_ref[i, 2]
        th = delta_ref[i, 3]
        x = tx * wa + xa
        y = ty * ha + ya
        w = jnp.exp(tw) * wa
        h = jnp.exp(th) * ha
        out_ref[i, 0] = jnp.clip(x - w * 0.5, 0.0, 1.0)
        out_ref[i, 1] = jnp.clip(x + w * 0.5, 0.0, 1.0)
        out_ref[i, 2] = jnp.clip(y - h * 0.5, 0.0, 1.0)
        out_ref[i, 3] = jnp.clip(y + h * 0.5, 0.0, 1.0)


def _make_sc_nms(N):
    G1 = N // L
    NV1 = G1 // L
    NV2 = (NV1 + L - 1) // L

    mesh = plsc.VectorSubcoreMesh(core_axis_name="c", subcore_axis_name="s")

    @functools.partial(
        pl.kernel, mesh=mesh,
        compiler_params=pltpu.CompilerParams(needs_layout_passes=False),
        out_type=jax.ShapeDtypeStruct((2 * 4 * KPAD,), jnp.float32),
        scratch_types=[
            pltpu.VMEM((N,), jnp.float32),
            pltpu.VMEM((N,), jnp.float32),
            pltpu.VMEM((N,), jnp.float32),
            pltpu.VMEM((N,), jnp.float32),
            pltpu.VMEM((N,), jnp.float32),
            pltpu.VMEM((G1,), jnp.float32),
            pltpu.VMEM((LANES,), jnp.float32),
            pltpu.VMEM((KPAD,), jnp.float32),
            pltpu.VMEM((KPAD,), jnp.float32),
            pltpu.VMEM((KPAD,), jnp.float32),
            pltpu.VMEM((KPAD,), jnp.float32),
        ],
    )
    def sc_nms(scores_hbm, boxes_hbm, out_hbm,
               sc_v, bx0_v, bx1_v, by0_v, by1_v, l1_v, l2_v,
               k0_v, k1_v, k2_v, k3_v):
        cid = lax.axis_index("c")
        sid = lax.axis_index("s")

        lanes = jnp.arange(L, dtype=jnp.int32)
        m0 = lanes == 0
        zf = jnp.zeros((L,), jnp.float32)
        zi = jnp.zeros((L,), jnp.int32)
        BIG = jnp.int32(1 << 24)

        def rot(x, sh):
            return x.at[(lanes + sh) % L].get(mode="promise_in_bounds")

        def allmax(x):
            for sh in (1, 2, 4, 8):
                x = jnp.maximum(x, rot(x, sh))
            return x

        def allmin(x):
            for sh in (1, 2, 4, 8):
                x = jnp.minimum(x, rot(x, sh))
            return x

        @pl.when(sid == 0)
        def _():
            i = cid
            pltpu.sync_copy(scores_hbm.at[pl.ds(i * N, N)], sc_v)
            pltpu.sync_copy(boxes_hbm.at[pl.ds((i * 4 + 0) * N, N)], bx0_v)
            pltpu.sync_copy(boxes_hbm.at[pl.ds((i * 4 + 1) * N, N)], bx1_v)
            pltpu.sync_copy(boxes_hbm.at[pl.ds((i * 4 + 2) * N, N)], by0_v)
            pltpu.sync_copy(boxes_hbm.at[pl.ds((i * 4 + 3) * N, N)], by1_v)

            def l1_body(cc, _):
                base = cc * L
                acc = zf - 1.0
                for r in range(L):
                    off = r * G1 + base
                    v = sc_v[pl.ds(off, L)]
                    v = jnp.where(v > PROP_T, v, -1.0)
                    sc_v[pl.ds(off, L)] = v
                    acc = jnp.maximum(acc, v)
                l1_v[pl.ds(base, L)] = acc
                return 0
            lax.fori_loop(0, NV1, l1_body, 0)

            for vi in range(LANES // L):
                l2_v[pl.ds(vi * L, L)] = zf - 1e9

            def l2_body(h, _):
                msp = allmax(l1_v[pl.ds(h * L, L)])
                plsc.store_scatter(l2_v, [zi + h], msp, mask=m0)
                return 0
            lax.fori_loop(0, NV1, l2_body, 0)

            for t in range(KPAD // L):
                k0_v[pl.ds(t * L, L)] = zf
                k1_v[pl.ds(t * L, L)] = zf
                k2_v[pl.ds(t * L, L)] = zf
                k3_v[pl.ds(t * L, L)] = zf

            def topmax():
                acc = l2_v[pl.ds(0, L)]
                for vi in range(1, NV2):
                    acc = jnp.maximum(acc, l2_v[pl.ds(vi * L, L)])
                return allmax(acc)

            def step(k_sp, nk):
                M = topmax()
                alive = jnp.logical_and(k_sp < MAX_OUT, M > SCORE_T)

                hc = zi + BIG
                for vi in range(NV2):
                    mvi = l2_v[pl.ds(vi * L, L)] == M
                    hc = jnp.minimum(hc, jnp.where(mvi, vi * L + lanes, BIG))
                h = allmin(hc)
                g1 = plsc.load_gather(l1_v, [h * L + lanes])
                c = allmin(jnp.where(g1 == M, h * L + lanes, BIG))
                col = plsc.load_gather(sc_v, [lanes * G1 + c])
                r = allmin(jnp.where(col == M, lanes, BIG))
                j = c * L + r

                wmask = jnp.logical_and(m0, alive)
                plsc.store_scatter(sc_v, [r * G1 + c], zf - 1.0, mask=wmask)
                newc = allmax(plsc.load_gather(sc_v, [lanes * G1 + c]))
                plsc.store_scatter(l1_v, [c], newc, mask=wmask)
                newh = allmax(plsc.load_gather(l1_v, [h * L + lanes]))
                plsc.store_scatter(l2_v, [h], newh, mask=wmask)

                x0 = plsc.load_gather(bx0_v, [j])
                x1 = plsc.load_gather(bx1_v, [j])
                y0 = plsc.load_gather(by0_v, [j])
                y1 = plsc.load_gather(by1_v, [j])

                area = (x1 - x0) * (y1 - y0)

                def iou_body(t, acc):
                    b = t * L
                    kx0 = k0_v[pl.ds(b, L)]
                    kx1 = k1_v[pl.ds(b, L)]
                    ky0 = k2_v[pl.ds(b, L)]
                    ky1 = k3_v[pl.ds(b, L)]
                    iw = jnp.maximum(
                        jnp.minimum(x1, kx1) - jnp.maximum(x0, kx0), 0.0)
                    ih = jnp.maximum(
                        jnp.minimum(y1, ky1) - jnp.maximum(y0, ky0), 0.0)
                    inter = iw * ih
                    areas = (kx1 - kx0) * (ky1 - ky0)
                    iou = inter / (area + areas - inter + 1e-9)
                    return jnp.logical_or(acc, iou > IOU_T)

                acc = lax.fori_loop(0, nk, iou_body, lanes < 0)
                ov = allmax(jnp.where(acc, 1, 0))
                keep = jnp.logical_and(alive, ov == 0)

                kmask = jnp.logical_and(m0, keep)
                plsc.store_scatter(k0_v, [k_sp], x0, mask=kmask)
                plsc.store_scatter(k1_v, [k_sp], x1, mask=kmask)
                plsc.store_scatter(k2_v, [k_sp], y0, mask=kmask)
                plsc.store_scatter(k3_v, [k_sp], y1, mask=kmask)

                k_sp = k_sp + jnp.where(keep, 1, 0)
                return k_sp, jnp.logical_and(alive, k_sp < MAX_OUT)

            def cond(carry):
                _, flag = carry
                return flag > 0

            def body(carry):
                k_sp, _ = carry
                alive = m0
                nk = k_sp[0] // L + 2
                for _ in range(UNROLL):
                    k_sp, alive = step(k_sp, nk)
                return k_sp, jnp.where(alive, 1, 0)[0]

            lax.while_loop(cond, body, (zi, jnp.int32(1)))

            pltpu.sync_copy(k0_v, out_hbm.at[pl.ds((i * 4 + 0) * KPAD, KPAD)])
            pltpu.sync_copy(k1_v, out_hbm.at[pl.ds((i * 4 + 1) * KPAD, KPAD)])
            pltpu.sync_copy(k2_v, out_hbm.at[pl.ds((i * 4 + 2) * KPAD, KPAD)])
            pltpu.sync_copy(k3_v, out_hbm.at[pl.ds((i * 4 + 3) * KPAD, KPAD)])

    return sc_nms


@jax.jit
def kernel(score_map, delta_map, anchors):
    B, H, W, A = score_map.shape
    N = H * W * A
    R = N // LANES
    assert N % LANES == 0 and B == 2

    deltas = delta_map.reshape(B, N, 4).transpose(0, 2, 1).reshape(B, 4, R, LANES)
    anch = anchors.reshape(N, 4).T.reshape(4, R, LANES)

    boxes = pl.pallas_call(
        _decode_body,
        out_shape=jax.ShapeDtypeStruct((B, 4, R, LANES), jnp.float32),
    )(deltas, anch)

    scores_t = (score_map.reshape(B, N // L, L)
                .transpose(0, 2, 1).reshape(B * N))
    boxes_flat = boxes.reshape(B * 4 * N)

    out = _make_sc_nms(N)(scores_t, boxes_flat)
    kept = out.reshape(B, 4, KPAD)[:, :, :MAX_OUT]
    return kept.transpose(0, 2, 1)
